# baseline (speedup 1.0000x reference)
import functools

import jax
import jax.numpy as jnp
from jax import lax
from jax.experimental import pallas as pl

B, T, R = 16, 512, 8192
K = 204
K2 = 5
RB_A = 1024
RB_C = 2048


def _pass_a_kernel(x_ref, wp_ref, bp_ref, border_ref, score_ref, inv_ref):
    xb = x_ref[0]
    nanm = jnp.isnan(xb)
    xc = jnp.where(nanm, 0.0, xb)
    border_ref[0] = nanm
    ss = jnp.sum(xc * xc, axis=0)
    norm = jnp.maximum(jnp.sqrt(ss), 1e-12)
    xn = (xc / norm[None, :]).astype(jnp.bfloat16)
    wb = wp_ref[...].astype(jnp.bfloat16)

    def dotc(lo):
        return jnp.dot(wb[:, lo:lo + 128], xn[lo:lo + 128, :],
                       preferred_element_type=jnp.float32)

    dt = (((dotc(384) + dotc(256)) + dotc(128)) + dotc(0))[0]
    score = dt + bp_ref[0, 0]
    anynan = jnp.any(nanm, axis=0)
    score = jnp.where(anynan, -10000.0, score)
    score_ref[0, 0] = score
    inv_ref[0, 0] = 1.0 / norm


def _topk_kernel(score_ref, idx_ref, row_ref):
    s0 = score_ref[...]
    iota_r = lax.broadcasted_iota(jnp.int32, (B, R), 1)
    iota_k = lax.broadcasted_iota(jnp.int32, (B, K), 1)

    def body(k, carry):
        s, row, idx_acc = carry
        m = jnp.max(s, axis=1, keepdims=True)
        eq = s == m
        idxv = jnp.min(jnp.where(eq, iota_r, R), axis=1, keepdims=True)
        hit = iota_r == idxv
        s = jnp.where(hit, -jnp.inf, s)
        row = jnp.where(hit, 1.0, row)
        idx_acc = jnp.where(iota_k == k, idxv, idx_acc)
        return s, row, idx_acc

    s, row, idx_acc = lax.fori_loop(
        0, K, body,
        (s0, jnp.zeros((B, R), jnp.float32), jnp.zeros((B, K), jnp.int32)))
    idx_ref[...] = idx_acc
    row_ref[...] = row


def _pass_c_kernel(x_ref, idx_ref, inv_ref, row_ref, mask_ref, xt_ref,
                  invsel_ref):
    nr = pl.program_id(1)
    xb = x_ref[0]
    xc = jnp.where(jnp.isnan(xb), 0.0, xb)
    invb = inv_ref[0, 0]
    rowb = row_ref[0, 0]
    mask_ref[0] = jnp.broadcast_to(rowb[None, :], (T, RB_C))
    idxv = idx_ref[0, 0]
    offs = nr * RB_C
    iota_rb = lax.broadcasted_iota(jnp.int32, (RB_C, K), 0) + offs
    sel = iota_rb == idxv[None, :]
    g01 = sel.astype(jnp.bfloat16)
    xh = xc.astype(jnp.bfloat16)
    xl = (xc - xh.astype(jnp.float32)).astype(jnp.bfloat16)
    part = (jnp.dot(xh, g01, preferred_element_type=jnp.float32)
            + jnp.dot(xl, g01, preferred_element_type=jnp.float32))
    ih = invb.astype(jnp.bfloat16)
    il = (invb - ih.astype(jnp.float32)).astype(jnp.bfloat16)
    ipart = (jnp.dot(ih[None, :], g01, preferred_element_type=jnp.float32)
             + jnp.dot(il[None, :], g01, preferred_element_type=jnp.float32))

    @pl.when(nr == 0)
    def _():
        xt_ref[0] = jnp.zeros((T, K), jnp.float32)
        invsel_ref[0] = jnp.zeros((1, K), jnp.float32)

    xt_ref[0] += part
    invsel_ref[0] += ipart


def _tail_kernel(xt_ref, invsel_ref, w1t_ref, b1_ref, gamma_ref, beta_ref,
                 w2t_ref, b2_ref, out_ref, hs_ref):
    w1t = w1t_ref[...]
    b1 = b1_ref[...]
    s = jnp.zeros((T, 1), jnp.float32)
    ss = jnp.zeros((T, 1), jnp.float32)
    for b in range(B):
        xtb = xt_ref[b] * invsel_ref[b]
        h = jnp.dot(xtb, w1t, preferred_element_type=jnp.float32) + b1
        hs_ref[b] = h
        s = s + jnp.sum(h, axis=1, keepdims=True)
        ss = ss + jnp.sum(h * h, axis=1, keepdims=True)
    n = float(B * K)
    mean = s / n
    var = ss / n - mean * mean
    a = gamma_ref[...] * lax.rsqrt(var + 1e-5)
    bterm = beta_ref[...] - mean * a
    w2t = w2t_ref[...]
    b2 = b2_ref[...]
    for b in range(B):
        hn = hs_ref[b] * a + bterm
        out_ref[b] = jnp.dot(hn, w2t, preferred_element_type=jnp.float32) + b2


@jax.jit
def kernel(x, Wp, bp, W1, b1, gamma, beta, W2, b2):
    nra = R // RB_A
    border, score3, inv3 = pl.pallas_call(
        _pass_a_kernel,
        grid=(B, nra),
        in_specs=[
            pl.BlockSpec((1, T, RB_A), lambda b, r: (b, 0, r)),
            pl.BlockSpec((1, T), lambda b, r: (0, 0)),
            pl.BlockSpec((1, 1), lambda b, r: (0, 0)),
        ],
        out_specs=[
            pl.BlockSpec((1, T, RB_A), lambda b, r: (b, 0, r)),
            pl.BlockSpec((1, 1, RB_A), lambda b, r: (b, 0, r)),
            pl.BlockSpec((1, 1, RB_A), lambda b, r: (b, 0, r)),
        ],
        out_shape=[
            jax.ShapeDtypeStruct((B, T, R), jnp.bool_),
            jax.ShapeDtypeStruct((B, 1, R), jnp.float32),
            jax.ShapeDtypeStruct((B, 1, R), jnp.float32),
        ],
    )(x, Wp, bp.reshape(1, 1))

    score = score3.reshape(B, R)
    topk_index, row = pl.pallas_call(
        _topk_kernel,
        out_shape=[
            jax.ShapeDtypeStruct((B, K), jnp.int32),
            jax.ShapeDtypeStruct((B, R), jnp.float32),
        ],
    )(score)

    nrc = R // RB_C
    mask, xt, invsel = pl.pallas_call(
        _pass_c_kernel,
        grid=(B, nrc),
        in_specs=[
            pl.BlockSpec((1, T, RB_C), lambda b, r: (b, 0, r)),
            pl.BlockSpec((1, 1, K), lambda b, r: (b, 0, 0)),
            pl.BlockSpec((1, 1, RB_C), lambda b, r: (b, 0, r)),
            pl.BlockSpec((1, 1, RB_C), lambda b, r: (b, 0, r)),
        ],
        out_specs=[
            pl.BlockSpec((1, T, RB_C), lambda b, r: (b, 0, r)),
            pl.BlockSpec((1, T, K), lambda b, r: (b, 0, 0)),
            pl.BlockSpec((1, 1, K), lambda b, r: (b, 0, 0)),
        ],
        out_shape=[
            jax.ShapeDtypeStruct((B, T, R), jnp.float32),
            jax.ShapeDtypeStruct((B, T, K), jnp.float32),
            jax.ShapeDtypeStruct((B, 1, K), jnp.float32),
        ],
    )(x, topk_index.reshape(B, 1, K), inv3, row.reshape(B, 1, R))

    out = pl.pallas_call(
        _tail_kernel,
        out_shape=jax.ShapeDtypeStruct((B, T, K2), jnp.float32),
        scratch_shapes=[pltpu_vmem((B, T, K), jnp.float32)],
    )(xt, invsel, W1.T, b1.reshape(1, K), gamma.reshape(T, 1),
      beta.reshape(T, 1), W2.T, b2.reshape(1, K2))

    return out, mask, border, topk_index


def pltpu_vmem(shape, dtype):
    from jax.experimental.pallas import tpu as pltpu
    return pltpu.VMEM(shape, dtype)


# speedup vs baseline: 1.0987x; 1.0987x over previous
import functools

import jax
import jax.numpy as jnp
from jax import lax
from jax.experimental import pallas as pl

B, T, R = 16, 512, 8192
K = 204
K2 = 5
RB_A = 2048
RB_C = 4096


def _pass_a_kernel(x_ref, wp_ref, bp_ref, border_ref, score_ref, inv_ref):
    xb = x_ref[0]
    nanm = jnp.isnan(xb)
    xc = jnp.where(nanm, 0.0, xb)
    border_ref[0] = nanm
    ss = jnp.sum(xc * xc, axis=0)
    norm = jnp.maximum(jnp.sqrt(ss), 1e-12)
    xn = (xc / norm[None, :]).astype(jnp.bfloat16)
    wb = wp_ref[...].astype(jnp.bfloat16)

    def dotc(lo):
        return jnp.dot(wb[:, lo:lo + 128], xn[lo:lo + 128, :],
                       preferred_element_type=jnp.float32)

    dt = (((dotc(384) + dotc(256)) + dotc(128)) + dotc(0))[0]
    score = dt + bp_ref[0, 0]
    anynan = jnp.any(nanm, axis=0)
    score = jnp.where(anynan, -10000.0, score)
    score_ref[0, 0] = score
    inv_ref[0, 0] = 1.0 / norm


def _topk_kernel(score_ref, idx_ref, row_ref):
    s0 = score_ref[...]
    iota_r = lax.broadcasted_iota(jnp.int32, (B, R), 1)
    iota_k = lax.broadcasted_iota(jnp.int32, (B, K), 1)

    def body(k, carry):
        s, row, idx_acc = carry
        m = jnp.max(s, axis=1, keepdims=True)
        eq = s == m
        idxv = jnp.min(jnp.where(eq, iota_r, R), axis=1, keepdims=True)
        hit = iota_r == idxv
        s = jnp.where(hit, -jnp.inf, s)
        row = jnp.where(hit, 1.0, row)
        idx_acc = jnp.where(iota_k == k, idxv, idx_acc)
        return s, row, idx_acc

    s, row, idx_acc = lax.fori_loop(
        0, K, body,
        (s0, jnp.zeros((B, R), jnp.float32), jnp.zeros((B, K), jnp.int32)))
    idx_ref[...] = idx_acc
    row_ref[...] = row


def _pass_c_kernel(x_ref, idx_ref, inv_ref, row_ref, mask_ref, xt_ref,
                  invsel_ref):
    nr = pl.program_id(1)
    xb = x_ref[0]
    xc = jnp.where(jnp.isnan(xb), 0.0, xb)
    invb = inv_ref[0, 0]
    rowb = row_ref[0, 0]
    mask_ref[0] = jnp.broadcast_to(rowb[None, :], (T, RB_C))
    idxv = idx_ref[0, 0]
    offs = nr * RB_C
    iota_rb = lax.broadcasted_iota(jnp.int32, (RB_C, K), 0) + offs
    sel = iota_rb == idxv[None, :]
    g01 = sel.astype(jnp.bfloat16)
    part = jnp.dot(xc.astype(jnp.bfloat16), g01,
                   preferred_element_type=jnp.float32)
    ih = invb.astype(jnp.bfloat16)
    il = (invb - ih.astype(jnp.float32)).astype(jnp.bfloat16)
    ipart = (jnp.dot(ih[None, :], g01, preferred_element_type=jnp.float32)
             + jnp.dot(il[None, :], g01, preferred_element_type=jnp.float32))

    @pl.when(nr == 0)
    def _():
        xt_ref[0] = jnp.zeros((T, K), jnp.float32)
        invsel_ref[0] = jnp.zeros((1, K), jnp.float32)

    xt_ref[0] += part
    invsel_ref[0] += ipart


def _tail_kernel(xt_ref, invsel_ref, w1t_ref, b1_ref, gamma_ref, beta_ref,
                 w2t_ref, b2_ref, out_ref, hs_ref):
    w1t = w1t_ref[...]
    b1 = b1_ref[...]
    s = jnp.zeros((T, 1), jnp.float32)
    ss = jnp.zeros((T, 1), jnp.float32)
    for b in range(B):
        xtb = xt_ref[b] * invsel_ref[b]
        h = jnp.dot(xtb, w1t, preferred_element_type=jnp.float32) + b1
        hs_ref[b] = h
        s = s + jnp.sum(h, axis=1, keepdims=True)
        ss = ss + jnp.sum(h * h, axis=1, keepdims=True)
    n = float(B * K)
    mean = s / n
    var = ss / n - mean * mean
    a = gamma_ref[...] * lax.rsqrt(var + 1e-5)
    bterm = beta_ref[...] - mean * a
    w2t = w2t_ref[...]
    b2 = b2_ref[...]
    for b in range(B):
        hn = hs_ref[b] * a + bterm
        out_ref[b] = jnp.dot(hn, w2t, preferred_element_type=jnp.float32) + b2


@jax.jit
def kernel(x, Wp, bp, W1, b1, gamma, beta, W2, b2):
    nra = R // RB_A
    border, score3, inv3 = pl.pallas_call(
        _pass_a_kernel,
        grid=(B, nra),
        in_specs=[
            pl.BlockSpec((1, T, RB_A), lambda b, r: (b, 0, r)),
            pl.BlockSpec((1, T), lambda b, r: (0, 0)),
            pl.BlockSpec((1, 1), lambda b, r: (0, 0)),
        ],
        out_specs=[
            pl.BlockSpec((1, T, RB_A), lambda b, r: (b, 0, r)),
            pl.BlockSpec((1, 1, RB_A), lambda b, r: (b, 0, r)),
            pl.BlockSpec((1, 1, RB_A), lambda b, r: (b, 0, r)),
        ],
        out_shape=[
            jax.ShapeDtypeStruct((B, T, R), jnp.bool_),
            jax.ShapeDtypeStruct((B, 1, R), jnp.float32),
            jax.ShapeDtypeStruct((B, 1, R), jnp.float32),
        ],
    )(x, Wp, bp.reshape(1, 1))

    score = score3.reshape(B, R)
    topk_index, row = pl.pallas_call(
        _topk_kernel,
        out_shape=[
            jax.ShapeDtypeStruct((B, K), jnp.int32),
            jax.ShapeDtypeStruct((B, R), jnp.float32),
        ],
    )(score)

    nrc = R // RB_C
    mask, xt, invsel = pl.pallas_call(
        _pass_c_kernel,
        grid=(B, nrc),
        in_specs=[
            pl.BlockSpec((1, T, RB_C), lambda b, r: (b, 0, r)),
            pl.BlockSpec((1, 1, K), lambda b, r: (b, 0, 0)),
            pl.BlockSpec((1, 1, RB_C), lambda b, r: (b, 0, r)),
            pl.BlockSpec((1, 1, RB_C), lambda b, r: (b, 0, r)),
        ],
        out_specs=[
            pl.BlockSpec((1, T, RB_C), lambda b, r: (b, 0, r)),
            pl.BlockSpec((1, T, K), lambda b, r: (b, 0, 0)),
            pl.BlockSpec((1, 1, K), lambda b, r: (b, 0, 0)),
        ],
        out_shape=[
            jax.ShapeDtypeStruct((B, T, R), jnp.float32),
            jax.ShapeDtypeStruct((B, T, K), jnp.float32),
            jax.ShapeDtypeStruct((B, 1, K), jnp.float32),
        ],
    )(x, topk_index.reshape(B, 1, K), inv3, row.reshape(B, 1, R))

    out = pl.pallas_call(
        _tail_kernel,
        out_shape=jax.ShapeDtypeStruct((B, T, K2), jnp.float32),
        scratch_shapes=[pltpu_vmem((B, T, K), jnp.float32)],
    )(xt, invsel, W1.T, b1.reshape(1, K), gamma.reshape(T, 1),
      beta.reshape(T, 1), W2.T, b2.reshape(1, K2))

    return out, mask, border, topk_index


def pltpu_vmem(shape, dtype):
    from jax.experimental.pallas import tpu as pltpu
    return pltpu.VMEM(shape, dtype)


# speedup vs baseline: 1.1248x; 1.0238x over previous
import functools

import jax
import jax.numpy as jnp
from jax import lax
from jax.experimental import pallas as pl

B, T, R = 16, 512, 8192
K = 204
K2 = 5
RB_A = 4096
RB_C = 4096


def _pass_a_kernel(x_ref, wp_ref, bp_ref, border_ref, score_ref, inv_ref):
    xb = x_ref[0]
    nanm = jnp.isnan(xb)
    xc = jnp.where(nanm, 0.0, xb)
    border_ref[0] = nanm
    ss = jnp.sum(xc * xc, axis=0)
    norm = jnp.maximum(jnp.sqrt(ss), 1e-12)
    xn = (xc / norm[None, :]).astype(jnp.bfloat16)
    wb = wp_ref[...].astype(jnp.bfloat16)

    def dotc(lo):
        return jnp.dot(wb[:, lo:lo + 128], xn[lo:lo + 128, :],
                       preferred_element_type=jnp.float32)

    dt = (((dotc(384) + dotc(256)) + dotc(128)) + dotc(0))[0]
    score = dt + bp_ref[0, 0]
    anynan = jnp.any(nanm, axis=0)
    score = jnp.where(anynan, -10000.0, score)
    score_ref[0, 0] = score
    inv_ref[0, 0] = 1.0 / norm


def _topk_kernel(score_ref, idx_ref, row_ref):
    s0 = score_ref[...]
    iota_r = lax.broadcasted_iota(jnp.int32, (B, R), 1)
    iota_k = lax.broadcasted_iota(jnp.int32, (B, K), 1)

    def body(k, carry):
        s, row, idx_acc = carry
        m = jnp.max(s, axis=1, keepdims=True)
        eq = s == m
        idxv = jnp.min(jnp.where(eq, iota_r, R), axis=1, keepdims=True)
        hit = iota_r == idxv
        s = jnp.where(hit, -jnp.inf, s)
        row = jnp.where(hit, 1.0, row)
        idx_acc = jnp.where(iota_k == k, idxv, idx_acc)
        return s, row, idx_acc

    s, row, idx_acc = lax.fori_loop(
        0, K, body,
        (s0, jnp.zeros((B, R), jnp.float32), jnp.zeros((B, K), jnp.int32)))
    idx_ref[...] = idx_acc
    row_ref[...] = row


def _pass_c_kernel(x_ref, idx_ref, inv_ref, row_ref, mask_ref, xt_ref,
                  invsel_ref):
    nr = pl.program_id(1)
    xb = x_ref[0]
    xc = jnp.where(jnp.isnan(xb), 0.0, xb)
    invb = inv_ref[0, 0]
    rowb = row_ref[0, 0]
    mask_ref[0] = jnp.broadcast_to(rowb[None, :], (T, RB_C))
    idxv = idx_ref[0, 0]
    offs = nr * RB_C
    iota_rb = lax.broadcasted_iota(jnp.int32, (RB_C, K), 0) + offs
    sel = iota_rb == idxv[None, :]
    g01 = sel.astype(jnp.bfloat16)
    part = jnp.dot(xc.astype(jnp.bfloat16), g01,
                   preferred_element_type=jnp.float32)
    ih = invb.astype(jnp.bfloat16)
    il = (invb - ih.astype(jnp.float32)).astype(jnp.bfloat16)
    ipart = (jnp.dot(ih[None, :], g01, preferred_element_type=jnp.float32)
             + jnp.dot(il[None, :], g01, preferred_element_type=jnp.float32))

    @pl.when(nr == 0)
    def _():
        xt_ref[0] = jnp.zeros((T, K), jnp.float32)
        invsel_ref[0] = jnp.zeros((1, K), jnp.float32)

    xt_ref[0] += part
    invsel_ref[0] += ipart


def _tail_kernel(xt_ref, invsel_ref, w1t_ref, b1_ref, gamma_ref, beta_ref,
                 w2t_ref, b2_ref, out_ref, hs_ref):
    w1t = w1t_ref[...]
    b1 = b1_ref[...]
    s = jnp.zeros((T, 1), jnp.float32)
    ss = jnp.zeros((T, 1), jnp.float32)
    for b in range(B):
        xtb = xt_ref[b] * invsel_ref[b]
        h = jnp.dot(xtb, w1t, preferred_element_type=jnp.float32) + b1
        hs_ref[b] = h
        s = s + jnp.sum(h, axis=1, keepdims=True)
        ss = ss + jnp.sum(h * h, axis=1, keepdims=True)
    n = float(B * K)
    mean = s / n
    var = ss / n - mean * mean
    a = gamma_ref[...] * lax.rsqrt(var + 1e-5)
    bterm = beta_ref[...] - mean * a
    w2t = w2t_ref[...]
    b2 = b2_ref[...]
    for b in range(B):
        hn = hs_ref[b] * a + bterm
        out_ref[b] = jnp.dot(hn, w2t, preferred_element_type=jnp.float32) + b2


@jax.jit
def kernel(x, Wp, bp, W1, b1, gamma, beta, W2, b2):
    nra = R // RB_A
    border, score3, inv3 = pl.pallas_call(
        _pass_a_kernel,
        grid=(B, nra),
        in_specs=[
            pl.BlockSpec((1, T, RB_A), lambda b, r: (b, 0, r)),
            pl.BlockSpec((1, T), lambda b, r: (0, 0)),
            pl.BlockSpec((1, 1), lambda b, r: (0, 0)),
        ],
        out_specs=[
            pl.BlockSpec((1, T, RB_A), lambda b, r: (b, 0, r)),
            pl.BlockSpec((1, 1, RB_A), lambda b, r: (b, 0, r)),
            pl.BlockSpec((1, 1, RB_A), lambda b, r: (b, 0, r)),
        ],
        out_shape=[
            jax.ShapeDtypeStruct((B, T, R), jnp.bool_),
            jax.ShapeDtypeStruct((B, 1, R), jnp.float32),
            jax.ShapeDtypeStruct((B, 1, R), jnp.float32),
        ],
    )(x, Wp, bp.reshape(1, 1))

    score = score3.reshape(B, R)
    topk_index, row = pl.pallas_call(
        _topk_kernel,
        out_shape=[
            jax.ShapeDtypeStruct((B, K), jnp.int32),
            jax.ShapeDtypeStruct((B, R), jnp.float32),
        ],
    )(score)

    nrc = R // RB_C
    mask, xt, invsel = pl.pallas_call(
        _pass_c_kernel,
        grid=(B, nrc),
        in_specs=[
            pl.BlockSpec((1, T, RB_C), lambda b, r: (b, 0, r)),
            pl.BlockSpec((1, 1, K), lambda b, r: (b, 0, 0)),
            pl.BlockSpec((1, 1, RB_C), lambda b, r: (b, 0, r)),
            pl.BlockSpec((1, 1, RB_C), lambda b, r: (b, 0, r)),
        ],
        out_specs=[
            pl.BlockSpec((1, T, RB_C), lambda b, r: (b, 0, r)),
            pl.BlockSpec((1, T, K), lambda b, r: (b, 0, 0)),
            pl.BlockSpec((1, 1, K), lambda b, r: (b, 0, 0)),
        ],
        out_shape=[
            jax.ShapeDtypeStruct((B, T, R), jnp.float32),
            jax.ShapeDtypeStruct((B, T, K), jnp.float32),
            jax.ShapeDtypeStruct((B, 1, K), jnp.float32),
        ],
    )(x, topk_index.reshape(B, 1, K), inv3, row.reshape(B, 1, R))

    out = pl.pallas_call(
        _tail_kernel,
        out_shape=jax.ShapeDtypeStruct((B, T, K2), jnp.float32),
        scratch_shapes=[pltpu_vmem((B, T, K), jnp.float32)],
    )(xt, invsel, W1.T, b1.reshape(1, K), gamma.reshape(T, 1),
      beta.reshape(T, 1), W2.T, b2.reshape(1, K2))

    return out, mask, border, topk_index


def pltpu_vmem(shape, dtype):
    from jax.experimental.pallas import tpu as pltpu
    return pltpu.VMEM(shape, dtype)
